# baseline (speedup 1.0000x reference)
import jax
import jax.numpy as jnp
from jax import lax
from jax.experimental import pallas as pl
from jax.experimental.pallas import tpu as pltpu
from jax.experimental.pallas import tpu_sc as plsc

NC, NS, L = 2, 16, 16
NW = NC * NS
B, H, W, D = 8, 256, 256, 128
PER_SAMPLE = H * W
TOTAL = B * PER_SAMPLE

HS = 192
WPS = NW // B
CHUNK = HS * W // WPS
YR = HS // WPS
NCH = 4
CSZ = CHUNK // NCH
CROWS = CSZ // L

HB = 32
NBT = (H - HS) // HB
NBLK = B * NBT


def _sc_body(r3d_hbm, in2d_hbm, idx_hbm, out_hbm,
             idx16_v, idxbuf_v, gbuf_v, ybuf_v, acc_v, ysem, *gsems):
    c = lax.axis_index("c")
    s = lax.axis_index("s")
    wid = s * NC + c
    i = wid // WPS
    q = wid % WPS

    ycopy = pltpu.async_copy(
        in2d_hbm.at[pl.ds(i * H + q * YR, YR)], ybuf_v, ysem)

    pltpu.sync_copy(idx_hbm, idx16_v.at[pl.ds(0, 8)])
    v = idx16_v[...]
    idx_s = jnp.int32(0)
    for j in range(B):
        idx_s = jnp.where(i == j, v[j], idx_s)

    lanes = lax.broadcasted_iota(jnp.int32, (L,), 0)
    base = idx_s + lanes * D + (i * PER_SAMPLE + q * CHUNK) * D

    copies = []
    for k in range(NCH):
        def build(t, carry, off=k * CROWS):
            idxbuf_v[pl.ds((off + t) * L, L)] = base + (off + t) * (L * D)
            return carry
        lax.fori_loop(0, CROWS, build, 0, unroll=8)
        copies.append(pltpu.async_copy(
            r3d_hbm.at[idxbuf_v.at[pl.ds(k * CSZ, CSZ)]],
            gbuf_v.at[pl.ds(k * CSZ, CSZ)],
            gsems[k]))
    ycopy.wait()

    acc = jnp.zeros((L,), jnp.float32)
    for k in range(NCH):
        copies[k].wait()

        def red(t, a, off=k * CSZ):
            e = off + t * L
            d = (gbuf_v[pl.ds(e, L)]
                 - ybuf_v[e // W, pl.ds(e % W, L)])
            return a + d * d
        acc = lax.fori_loop(0, CROWS, red, acc, unroll=8)
    acc_v[...] = acc
    pltpu.sync_copy(acc_v, out_hbm.at[wid])


def _tc_body(idx_ref, x_ref, y_ref, out_ref):
    g = pl.program_id(0)
    i = g // NBT
    j = idx_ref[i]
    x = x_ref[0]
    sel = lax.broadcasted_iota(jnp.int32, (HB, W, D), 2) == j
    xs = jnp.sum(jnp.where(sel, x, 0.0), axis=2)
    d = xs - y_ref[0]

    @pl.when(g == 0)
    def _():
        out_ref[0, 0] = 0.0
    out_ref[0, 0] += jnp.sum(d * d)


def kernel(reconstructed_3d, input_2d, slice_idx):
    r3d_flat = reconstructed_3d.reshape(-1)
    in2d = input_2d.reshape(B * H, W)
    idx = slice_idx.astype(jnp.int32)

    mesh = plsc.VectorSubcoreMesh(core_axis_name="c", subcore_axis_name="s")
    partials = pl.kernel(
        _sc_body,
        out_type=jax.ShapeDtypeStruct((NW, L), jnp.float32),
        mesh=mesh,
        scratch_types=[
            pltpu.VMEM((L,), jnp.int32),
            pltpu.VMEM((CHUNK,), jnp.int32),
            pltpu.VMEM((CHUNK,), jnp.float32),
            pltpu.VMEM((YR, W), jnp.float32),
            pltpu.VMEM((L,), jnp.float32),
            pltpu.SemaphoreType.DMA,
        ] + [pltpu.SemaphoreType.DMA] * NCH,
    )(r3d_flat, in2d, idx)

    grid_spec = pltpu.PrefetchScalarGridSpec(
        num_scalar_prefetch=1,
        grid=(NBLK,),
        in_specs=[
            pl.BlockSpec((1, HB, W, D),
                         lambda g, idx_ref: (g // NBT, HS // HB + g % NBT, 0, 0)),
            pl.BlockSpec((1, HB, W),
                         lambda g, idx_ref: (g // NBT, HS // HB + g % NBT, 0)),
        ],
        out_specs=pl.BlockSpec(
            (1, 1), lambda g, idx_ref: (0, 0), memory_space=pltpu.SMEM),
    )
    tc_sum = pl.pallas_call(
        _tc_body,
        grid_spec=grid_spec,
        out_shape=jax.ShapeDtypeStruct((1, 1), jnp.float32),
    )(idx, reconstructed_3d, input_2d)

    return (jnp.sum(partials) + tc_sum[0, 0]) / TOTAL
